# baseline (speedup 1.0000x reference)
import functools

import jax
import jax.numpy as jnp
from jax import lax
from jax.experimental import pallas as pl
from jax.experimental.pallas import tpu as pltpu
from jax.experimental.pallas import tpu_sc as plsc

NC = 2
NS = 16
CH = 128
NB = 2
ZB = 64


@functools.cache
def _mesh():
    return plsc.VectorSubcoreMesh(core_axis_name="c", subcore_axis_name="s",
                                  num_cores=NC, num_subcores=NS)


def _pad_up(v, m):
    return (v + m - 1) // m * m


def _sc_degrees(srcr, dstr, n_pad):
    nch = srcr.shape[1]
    zr = n_pad // NS

    @functools.partial(
        pl.kernel,
        out_type=jax.ShapeDtypeStruct((NC, 2, n_pad), jnp.float32),
        mesh=_mesh(),
        scratch_types=[
            pltpu.VMEM((nch, CH), jnp.int32),
            pltpu.VMEM((nch, CH), jnp.int32),
            pltpu.VMEM((CH,), jnp.float32),
            pltpu.VMEM((zr,), jnp.float32),
            pltpu.VMEM_SHARED((n_pad,), jnp.float32),
            pltpu.VMEM_SHARED((n_pad,), jnp.float32),
        ],
    )
    def deg_kernel(src_hbm, dst_hbm, out_hbm, idx_s, idx_d, ones, zb,
                   dego, degi):
        cid = lax.axis_index("c")
        sid = lax.axis_index("s")
        w = cid * NS + sid
        pltpu.sync_copy(src_hbm.at[w], idx_s)
        pltpu.sync_copy(dst_hbm.at[w], idx_d)

        def fill(i, _):
            ones[pl.ds(i * 16, 16)] = jnp.ones((16,), jnp.float32)
            return 0

        lax.fori_loop(0, CH // 16, fill, 0)

        def zfill(i, _):
            zb[pl.ds(i * 16, 16)] = jnp.zeros((16,), jnp.float32)
            return 0

        lax.fori_loop(0, zr // 16, zfill, 0)
        pltpu.sync_copy(zb, dego.at[pl.ds(sid * zr, zr)])
        pltpu.sync_copy(zb, degi.at[pl.ds(sid * zr, zr)])
        plsc.subcore_barrier()

        def body(j, _):
            pltpu.sync_copy(ones, dego.at[idx_s.at[j]], add=True)
            pltpu.sync_copy(ones, degi.at[idx_d.at[j]], add=True)
            return 0

        lax.fori_loop(0, nch, body, 0)
        plsc.subcore_barrier()
        sl = pl.ds(sid * zr, zr)
        pltpu.sync_copy(dego.at[sl], out_hbm.at[cid, 0, sl])
        pltpu.sync_copy(degi.at[sl], out_hbm.at[cid, 1, sl])

    return deg_kernel(srcr, dstr)


CA = 128


def _sc_aggregate(h, srcr, dstr, n, half, half_pad, feat, tc_tiling=None):
    nr = srcr.shape[1]
    ept = nr * CA
    zr = half_pad // NS
    dummy = half_pad - 1
    assert ept % CA == 0 and zr % CA == 0 and feat % 16 == 0

    @functools.partial(
        pl.kernel,
        out_type=jax.ShapeDtypeStruct((NC, half_pad, feat), jnp.float32),
        mesh=_mesh(),
        compiler_params=pltpu.CompilerParams(needs_layout_passes=False)
        if tc_tiling is None
        else pltpu.CompilerParams(use_tc_tiling_on_sc=tc_tiling,
                                  needs_layout_passes=False),
        scratch_types=[
            pltpu.VMEM((nr + 1, CA), jnp.int32),
            pltpu.VMEM((nr + 1, CA), jnp.int32),
            pltpu.VMEM((CA + 32, ), jnp.int32),
            pltpu.VMEM((CA + 32, ), jnp.int32),
            pltpu.VMEM((CA, feat), jnp.float32),
            pltpu.VMEM((CA, feat), jnp.float32),
            pltpu.VMEM_SHARED((half_pad, feat), jnp.float32),
            pltpu.SemaphoreType.DMA,
            pltpu.SemaphoreType.DMA,
        ],
    )
    def agg_kernel(h_hbm, src_hbm, dst_hbm, out_hbm, idx_s, idx_d,
                   st_s, st_d, r0, r1, acc, g0, g1):
        rows = (r0, r1)
        gsem = (g0, g1)
        cid = lax.axis_index("c")
        sid = lax.axis_index("s")
        pltpu.sync_copy(src_hbm.at[sid], idx_s.at[pl.ds(0, nr)])
        pltpu.sync_copy(dst_hbm.at[sid], idx_d.at[pl.ds(0, nr)])

        lo = (cid * half).astype(jnp.int32)
        sz = jnp.where(cid == 0, half, n - half).astype(jnp.int32)
        lane = lax.iota(jnp.int32, 16)

        def comp_row(i, carry):
            r, scnt = carry
            for q in range(CA // 16):
                vs = idx_s[i, pl.ds(q * 16, 16)]
                vd = idx_d[i, pl.ds(q * 16, 16)] - lo
                keep = jnp.logical_and(vd >= 0, vd < sz)
                ki = keep.astype(jnp.int32)
                pc = plsc.cumsum(ki)
                pos = pc - ki + scnt
                tgt = jnp.where(keep, pos, CA + 16 + lane)
                plsc.store_scatter(st_s, [tgt], vs)
                plsc.store_scatter(st_d, [tgt], jnp.where(keep, vd, dummy))
                scnt = scnt + lax.squeeze(lax.slice(pc, (15,), (16,)), (0,))
                full = scnt >= CA

                @pl.when(full)
                def _():
                    for q2 in range(CA // 16):
                        idx_s[r, pl.ds(q2 * 16, 16)] = st_s[pl.ds(q2 * 16, 16)]
                        idx_d[r, pl.ds(q2 * 16, 16)] = st_d[pl.ds(q2 * 16, 16)]
                    t0 = st_s[pl.ds(CA, 16)]
                    st_s[pl.ds(0, 16)] = t0
                    u0 = st_d[pl.ds(CA, 16)]
                    st_d[pl.ds(0, 16)] = u0

                fi = full.astype(jnp.int32)
                r = r + fi
                scnt = scnt - CA * fi
            return (r, scnt)

        r, scnt = lax.fori_loop(0, nr, comp_row,
                                (jnp.int32(0), jnp.int32(0)))
        for q2 in range(CA // 16):
            ln = lane + q2 * 16
            idx_s[r, pl.ds(q2 * 16, 16)] = jnp.where(
                ln < scnt, st_s[pl.ds(q2 * 16, 16)], 0)
            idx_d[r, pl.ds(q2 * 16, 16)] = jnp.where(
                ln < scnt, st_d[pl.ds(q2 * 16, 16)], dummy)
        nck = r + (scnt > 0).astype(jnp.int32)
        nk = nck
        for q2 in range(CA // 16):
            idx_s[nk, pl.ds(q2 * 16, 16)] = jnp.zeros((16,), jnp.int32)
            idx_d[nk, pl.ds(q2 * 16, 16)] = jnp.full((16,), dummy, jnp.int32)

        def zrow(i, _):
            for cc in range(feat // 16):
                rows[0][i, pl.ds(cc * 16, 16)] = jnp.zeros((16,), jnp.float32)
            return 0

        lax.fori_loop(0, CA, zrow, 0)
        for b in range(zr // CA):
            pltpu.sync_copy(rows[0], acc.at[pl.ds(sid * zr + b * CA, CA)])
        plsc.subcore_barrier()

        def group(g, _):
            descs = [
                pltpu.async_copy(h_hbm.at[idx_s.at[g * 2 + b]], rows[b], gsem[b])
                for b in range(2)
            ]
            for b in range(2):
                descs[b].wait()
                pltpu.sync_copy(rows[b], acc.at[idx_d.at[g * 2 + b]], add=True)
            return 0

        lax.fori_loop(0, (nck + 1) // 2, group, 0)
        plsc.subcore_barrier()
        sl = pl.ds(sid * zr, zr)
        pltpu.sync_copy(acc.at[sl], out_hbm.at[cid, sl])

    return agg_kernel(h, srcr, dstr)


_BLK = 1000


def _norm(deg2):
    return lax.rsqrt(jnp.maximum(deg2[:, 0] + deg2[:, 1], 1.0))


def _agg_spec(half, feat):
    npb = half // _BLK
    return pl.BlockSpec((1, _BLK, feat), lambda i: (i // npb, i % npb, 0))


def _tc1(x, w1, dego):
    n, f = x.shape

    def body(x_ref, w_ref, dg_ref, h_ref):
        ns = _norm(dg_ref[...])
        h = jnp.dot(x_ref[...], w_ref[...],
                    preferred_element_type=jnp.float32,
                    precision=lax.Precision.HIGHEST)
        h_ref[...] = h * ns[:, None]

    return pl.pallas_call(
        body,
        grid=(n // _BLK,),
        in_specs=[
            pl.BlockSpec((_BLK, f), lambda i: (i, 0)),
            pl.BlockSpec((f, f), lambda i: (0, 0)),
            pl.BlockSpec((_BLK, 2), lambda i: (i, 0)),
        ],
        out_specs=pl.BlockSpec((_BLK, f), lambda i: (i, 0)),
        out_shape=jax.ShapeDtypeStruct((n, f), jnp.float32),
    )(x, w1, dego)


def _tc2(agg1, dego, degi, b1, w2p, n, half):
    f = agg1.shape[2]
    cp = w2p.shape[1]

    def body(a_ref, dgo_ref, dgi_ref, b1_ref, w2_ref, z_ref):
        nd = _norm(dgi_ref[...])
        ns = _norm(dgo_ref[...])
        h2 = a_ref[0] * nd[:, None] + b1_ref[...]
        h2 = jnp.maximum(h2, 0.0) * ns[:, None]
        z_ref[...] = jnp.dot(h2, w2_ref[...],
                             preferred_element_type=jnp.float32,
                             precision=lax.Precision.HIGHEST)

    return pl.pallas_call(
        body,
        grid=(n // _BLK,),
        in_specs=[
            _agg_spec(half, f),
            pl.BlockSpec((_BLK, 2), lambda i: (i, 0)),
            pl.BlockSpec((_BLK, 2), lambda i: (i, 0)),
            pl.BlockSpec((1, f), lambda i: (0, 0)),
            pl.BlockSpec((f, cp), lambda i: (0, 0)),
        ],
        out_specs=pl.BlockSpec((_BLK, cp), lambda i: (i, 0)),
        out_shape=jax.ShapeDtypeStruct((n, cp), jnp.float32),
    )(agg1, dego, degi, b1, w2p)


def _tc3(agg2, degi, b2p, n, half):
    cp = agg2.shape[2]

    def body(a_ref, dgi_ref, b2_ref, o_ref):
        nd = _norm(dgi_ref[...])
        o_ref[...] = a_ref[0] * nd[:, None] + b2_ref[...]

    return pl.pallas_call(
        body,
        grid=(n // _BLK,),
        in_specs=[
            _agg_spec(half, cp),
            pl.BlockSpec((_BLK, 2), lambda i: (i, 0)),
            pl.BlockSpec((1, cp), lambda i: (0, 0)),
        ],
        out_specs=pl.BlockSpec((_BLK, cp), lambda i: (i, 0)),
        out_shape=jax.ShapeDtypeStruct((n, cp), jnp.float32),
    )(agg2, degi, b2p)


def kernel(x, edge_index, W1, b1, W2, b2):
    n, f = x.shape
    e = edge_index.shape[1]
    c = W2.shape[1]
    cp = _pad_up(c, 128)
    half = n // 2
    assert n % (2 * _BLK) == 0
    half_pad = _pad_up(half + 1, NS * CA)
    n_pad = _pad_up(n + 1, NS * CH)

    src = edge_index[0].astype(jnp.int32)
    dst = edge_index[1].astype(jnp.int32)
    e_pad = _pad_up(e, NC * NS * CH)
    src = jnp.concatenate([src, jnp.zeros((e_pad - e,), jnp.int32)])
    dst = jnp.concatenate([dst, jnp.full((e_pad - e,), n, jnp.int32)])
    nch32 = e_pad // (NC * NS * CH)
    srcr32 = src.reshape(NC * NS, nch32, CH)
    dstr32 = dst.reshape(NC * NS, nch32, CH)
    ept = e_pad // NS
    srcr16 = src.reshape(NS, ept // CA, CA)
    dstr16 = dst.reshape(NS, ept // CA, CA)

    degp = _sc_degrees(srcr32, dstr32, n_pad)
    dego = degp[:, 0, :].T
    degi = degp[:, 1, :].T

    h = _tc1(x, W1, dego)
    agg1 = _sc_aggregate(h, srcr16, dstr16, n, half, half_pad, f)

    w2p = jnp.zeros((f, cp), jnp.float32).at[:, :c].set(W2)
    b2p = jnp.zeros((1, cp), jnp.float32).at[0, :c].set(b2)
    z = _tc2(agg1, dego, degi, b1.reshape(1, f), w2p, n, half)
    agg2 = _sc_aggregate(z, srcr16, dstr16, n, half, half_pad, cp)

    out = _tc3(agg2, degi, b2p, n, half)
    return out[:, :c]


# speedup vs baseline: 1.2763x; 1.2763x over previous
import functools

import jax
import jax.numpy as jnp
from jax import lax
from jax.experimental import pallas as pl
from jax.experimental.pallas import tpu as pltpu
from jax.experimental.pallas import tpu_sc as plsc

NC = 2
NS = 16
CH = 128


@functools.cache
def _mesh():
    return plsc.VectorSubcoreMesh(core_axis_name="c", subcore_axis_name="s",
                                  num_cores=NC, num_subcores=NS)


def _pad_up(v, m):
    return (v + m - 1) // m * m


def _sc_degrees(srcr, dstr, n_pad):
    nch = srcr.shape[1]
    zr = n_pad // NS

    @functools.partial(
        pl.kernel,
        out_type=jax.ShapeDtypeStruct((NC, 2, n_pad), jnp.float32),
        mesh=_mesh(),
        scratch_types=[
            pltpu.VMEM((nch, CH), jnp.int32),
            pltpu.VMEM((nch, CH), jnp.int32),
            pltpu.VMEM((CH,), jnp.float32),
            pltpu.VMEM((zr,), jnp.float32),
            pltpu.VMEM_SHARED((n_pad,), jnp.float32),
            pltpu.VMEM_SHARED((n_pad,), jnp.float32),
        ],
    )
    def deg_kernel(src_hbm, dst_hbm, out_hbm, idx_s, idx_d, ones, zb,
                   dego, degi):
        cid = lax.axis_index("c")
        sid = lax.axis_index("s")
        w = cid * NS + sid
        pltpu.sync_copy(src_hbm.at[w], idx_s)
        pltpu.sync_copy(dst_hbm.at[w], idx_d)

        def fill(i, _):
            ones[pl.ds(i * 16, 16)] = jnp.ones((16,), jnp.float32)
            return 0

        lax.fori_loop(0, CH // 16, fill, 0)

        def zfill(i, _):
            zb[pl.ds(i * 16, 16)] = jnp.zeros((16,), jnp.float32)
            return 0

        lax.fori_loop(0, zr // 16, zfill, 0)
        pltpu.sync_copy(zb, dego.at[pl.ds(sid * zr, zr)])
        pltpu.sync_copy(zb, degi.at[pl.ds(sid * zr, zr)])
        plsc.subcore_barrier()

        def body(j, _):
            pltpu.sync_copy(ones, dego.at[idx_s.at[j]], add=True)
            pltpu.sync_copy(ones, degi.at[idx_d.at[j]], add=True)
            return 0

        lax.fori_loop(0, nch, body, 0)
        plsc.subcore_barrier()
        sl = pl.ds(sid * zr, zr)
        pltpu.sync_copy(dego.at[sl], out_hbm.at[cid, 0, sl])
        pltpu.sync_copy(degi.at[sl], out_hbm.at[cid, 1, sl])

    return deg_kernel(srcr, dstr)


def _sc_aggregate(h, srcr, dstr, n_pad, feat):
    nch = srcr.shape[1]
    zr = n_pad // NS
    nzb = zr // CH

    @functools.partial(
        pl.kernel,
        out_type=jax.ShapeDtypeStruct((NC, n_pad, feat), jnp.float32),
        mesh=_mesh(),
        scratch_types=[
            pltpu.VMEM((nch, CH), jnp.int32),
            pltpu.VMEM((nch, CH), jnp.int32),
            pltpu.VMEM((CH, feat), jnp.float32),
            pltpu.VMEM_SHARED((n_pad, feat), jnp.float32),
            pltpu.SemaphoreType.DMA,
        ],
    )
    def agg_kernel(h_hbm, src_hbm, dst_hbm, out_hbm, idx_s, idx_d, rows,
                   acc, sem):
        cid = lax.axis_index("c")
        sid = lax.axis_index("s")
        w = cid * NS + sid
        pltpu.sync_copy(src_hbm.at[w], idx_s)
        pltpu.sync_copy(dst_hbm.at[w], idx_d)

        def zrow(i, _):
            for cc in range(feat // 16):
                rows[i, pl.ds(cc * 16, 16)] = jnp.zeros((16,), jnp.float32)
            return 0

        lax.fori_loop(0, CH, zrow, 0)
        for b in range(nzb):
            pltpu.sync_copy(rows, acc.at[pl.ds(sid * zr + b * CH, CH)])
        plsc.subcore_barrier()

        def body(j, _):
            pltpu.async_copy(h_hbm.at[idx_s.at[j]], rows, sem).wait()
            pltpu.sync_copy(rows, acc.at[idx_d.at[j]], add=True)
            return 0

        lax.fori_loop(0, nch, body, 0)
        plsc.subcore_barrier()
        sl = pl.ds(sid * zr, zr)
        pltpu.sync_copy(acc.at[sl], out_hbm.at[cid, sl])

    return agg_kernel(h, srcr, dstr)


_BLK = 2000


def _norm(deg2):
    return lax.rsqrt(jnp.maximum(deg2[:, 0] + deg2[:, 1], 1.0))


def _tc1(x, w1, dego):
    n, f = x.shape

    def body(x_ref, w_ref, dg_ref, h_ref):
        ns = _norm(dg_ref[...])
        h = jnp.dot(x_ref[...], w_ref[...],
                    preferred_element_type=jnp.float32,
                    precision=lax.Precision.HIGHEST)
        h_ref[...] = h * ns[:, None]

    return pl.pallas_call(
        body,
        grid=(n // _BLK,),
        in_specs=[
            pl.BlockSpec((_BLK, f), lambda i: (i, 0)),
            pl.BlockSpec((f, f), lambda i: (0, 0)),
            pl.BlockSpec((_BLK, 2), lambda i: (i, 0)),
        ],
        out_specs=pl.BlockSpec((_BLK, f), lambda i: (i, 0)),
        out_shape=jax.ShapeDtypeStruct((n, f), jnp.float32),
    )(x, w1, dego)


def _tc2(agg1, dego, degi, b1, w2p, n):
    f = agg1.shape[2]
    cp = w2p.shape[1]

    def body(a_ref, dgo_ref, dgi_ref, b1_ref, w2_ref, z_ref):
        nd = _norm(dgi_ref[...])
        ns = _norm(dgo_ref[...])
        h2 = (a_ref[0] + a_ref[1]) * nd[:, None] + b1_ref[...]
        h2 = jnp.maximum(h2, 0.0) * ns[:, None]
        z_ref[...] = jnp.dot(h2, w2_ref[...],
                             preferred_element_type=jnp.float32,
                             precision=lax.Precision.HIGHEST)

    return pl.pallas_call(
        body,
        grid=(n // _BLK,),
        in_specs=[
            pl.BlockSpec((NC, _BLK, f), lambda i: (0, i, 0)),
            pl.BlockSpec((_BLK, 2), lambda i: (i, 0)),
            pl.BlockSpec((_BLK, 2), lambda i: (i, 0)),
            pl.BlockSpec((1, f), lambda i: (0, 0)),
            pl.BlockSpec((f, cp), lambda i: (0, 0)),
        ],
        out_specs=pl.BlockSpec((_BLK, cp), lambda i: (i, 0)),
        out_shape=jax.ShapeDtypeStruct((n, cp), jnp.float32),
    )(agg1, dego, degi, b1, w2p)


def _tc3(agg2, degi, b2p, n):
    cp = agg2.shape[2]

    def body(a_ref, dgi_ref, b2_ref, o_ref):
        nd = _norm(dgi_ref[...])
        o_ref[...] = (a_ref[0] + a_ref[1]) * nd[:, None] + b2_ref[...]

    return pl.pallas_call(
        body,
        grid=(n // _BLK,),
        in_specs=[
            pl.BlockSpec((NC, _BLK, cp), lambda i: (0, i, 0)),
            pl.BlockSpec((_BLK, 2), lambda i: (i, 0)),
            pl.BlockSpec((1, cp), lambda i: (0, 0)),
        ],
        out_specs=pl.BlockSpec((_BLK, cp), lambda i: (i, 0)),
        out_shape=jax.ShapeDtypeStruct((n, cp), jnp.float32),
    )(agg2, degi, b2p)


def kernel(x, edge_index, W1, b1, W2, b2):
    n, f = x.shape
    e = edge_index.shape[1]
    c = W2.shape[1]
    cp = _pad_up(c, 128)
    n_pad = _pad_up(n + 1, NS * CH)

    src = edge_index[0].astype(jnp.int32)
    dst = edge_index[1].astype(jnp.int32)
    e_pad = _pad_up(e, NC * NS * CH)
    src = jnp.concatenate([src, jnp.zeros((e_pad - e,), jnp.int32)])
    dst = jnp.concatenate([dst, jnp.full((e_pad - e,), n, jnp.int32)])
    nch = e_pad // (NC * NS * CH)
    srcr = src.reshape(NC * NS, nch, CH)
    dstr = dst.reshape(NC * NS, nch, CH)

    degp = _sc_degrees(srcr, dstr, n_pad)
    dego = degp[:, 0, :].T
    degi = degp[:, 1, :].T

    h = _tc1(x, W1, dego)
    agg1 = _sc_aggregate(h, srcr, dstr, n_pad, f)

    w2p = jnp.zeros((f, cp), jnp.float32).at[:, :c].set(W2)
    b2p = jnp.zeros((1, cp), jnp.float32).at[0, :c].set(b2)
    z = _tc2(agg1, dego, degi, b1.reshape(1, f), w2p, n)
    agg2 = _sc_aggregate(z, srcr, dstr, n_pad, cp)

    out = _tc3(agg2, degi, b2p, n)
    return out[:, :c]
